# Initial kernel scaffold; baseline (speedup 1.0000x reference)
import jax
import jax.numpy as jnp
from jax.experimental import pallas as pl


def kernel(stories, query, E, candidates, A, W, H_w, H_b):
    raise NotImplementedError("write your pallas kernel here")


# speedup vs baseline: 19.0433x; 19.0433x over previous
import functools

import jax
import jax.numpy as jnp
from jax import lax
from jax.experimental import pallas as pl
from jax.experimental.pallas import tpu as pltpu
from jax.experimental.pallas import tpu_sc as plsc

B, M, S, C, V, D = 32, 50, 20, 1000, 100000, 64
HOPS = 3
NC, NS = 2, 16
NW = NC * NS
SEG_A = 52
SEG_W = 32
CHUNK = 4
CP = 1024
VP = 100352
VB = 2048

_mesh = plsc.VectorSubcoreMesh(
    core_axis_name="c", subcore_axis_name="s", num_cores=NC, num_subcores=NS)


def _wid():
  return lax.axis_index("s") * NC + lax.axis_index("c")


@functools.partial(
    pl.kernel,
    out_type=(
        jax.ShapeDtypeStruct((NW, SEG_A, D), jnp.float32),
        jax.ShapeDtypeStruct((NW, SEG_W, D), jnp.float32),
    ),
    mesh=_mesh,
    compiler_params=pltpu.CompilerParams(
        use_tc_tiling_on_sc=False, needs_layout_passes=False),
    scratch_types=[
        pltpu.VMEM((SEG_A * S,), jnp.int32),
        pltpu.VMEM((SEG_W * S,), jnp.int32),
        pltpu.VMEM((SEG_A * S, D), jnp.float32),
        pltpu.VMEM((SEG_A, D), jnp.float32),
        pltpu.VMEM((SEG_W, D), jnp.float32),
        pltpu.SemaphoreType.DMA,
    ],
)
def _sc_gather(idxA_hbm, idxW_hbm, A_hbm, W_hbm, outA_hbm, outW_hbm,
               idxA_v, idxW_v, rows_v, oA_v, oW_v, sem):
  w = _wid()
  pltpu.sync_copy(idxA_hbm.at[w], idxA_v)
  pltpu.sync_copy(idxW_hbm.at[w], idxW_v)

  def run_phase(table_hbm, idx_v, o_v, nseg):
    nchunk = nseg // CHUNK
    rows_per = CHUNK * S
    cps = [
        pltpu.async_copy(
            table_hbm.at[idx_v.at[pl.ds(c * rows_per, rows_per)]],
            rows_v.at[pl.ds(c * rows_per, rows_per)], sem)
        for c in range(nchunk)
    ]
    for cp in cps:
      cp.wait()

    def seg_body(i, _):
      base = i * S
      for d in range(D // 16):
        sl = pl.ds(d * 16, 16)
        acc = rows_v[base, sl]
        for r in range(1, S):
          acc = acc + rows_v[base + r, sl]
        o_v[i, sl] = acc
      return 0

    lax.fori_loop(0, nseg, seg_body, 0)

  run_phase(A_hbm, idxA_v, oA_v, SEG_A)
  pltpu.sync_copy(oA_v, outA_hbm.at[w])
  run_phase(W_hbm, idxW_v, oW_v, SEG_W)
  pltpu.sync_copy(oW_v, outW_hbm.at[w])


def _tc_hops_body(u0_ref, es_ref, Hw_ref, Hb_ref, Wc_ref, u_ref, csc_ref):
  u = u0_ref[...]
  es = es_ref[...]
  Hw = Hw_ref[...]
  Hb = Hb_ref[...]
  for _ in range(HOPS):
    sc = jnp.sum(es * u[:, None, :], axis=2)
    sc = sc - jnp.max(sc, axis=1, keepdims=True)
    e = jnp.exp(sc)
    att = e / jnp.sum(e, axis=1, keepdims=True)
    attn = jnp.sum(att[:, :, None] * es, axis=1)
    u = lax.dot_general(u, Hw, (((1,), (1,)), ((), ()))) + Hb + attn
  u_ref[...] = u
  csc_ref[...] = lax.dot_general(u, Wc_ref[...], (((1,), (1,)), ((), ())))


_tc_hops = pl.pallas_call(
    _tc_hops_body,
    out_shape=(
        jax.ShapeDtypeStruct((B, D), jnp.float32),
        jax.ShapeDtypeStruct((B, CP), jnp.float32),
    ),
)


def _tc_proj_body(u_ref, w_ref, o_ref):
  o_ref[...] = lax.dot_general(u_ref[...], w_ref[...],
                               (((1,), (1,)), ((), ())))


_tc_proj = pl.pallas_call(
    _tc_proj_body,
    grid=(VP // VB,),
    in_specs=[
        pl.BlockSpec((B, D), lambda i: (0, 0)),
        pl.BlockSpec((VB, D), lambda i: (i, 0)),
    ],
    out_specs=pl.BlockSpec((B, VB), lambda i: (0, i)),
    out_shape=jax.ShapeDtypeStruct((B, VP), jnp.float32),
)


@functools.partial(
    pl.kernel,
    out_type=jax.ShapeDtypeStruct((B, CP), jnp.float32),
    mesh=_mesh,
    compiler_params=pltpu.CompilerParams(
        use_tc_tiling_on_sc=False, needs_layout_passes=False),
    scratch_types=[
        pltpu.VMEM((VP,), jnp.float32),
        pltpu.VMEM((S * CP,), jnp.int32),
        pltpu.VMEM((CP,), jnp.float32),
        pltpu.SemaphoreType.DMA,
    ],
)
def _sc_score(proj_hbm, Et_hbm, csc_hbm, out_hbm, projv, idxv, outv, sem):
  b = _wid()
  pltpu.sync_copy(proj_hbm.at[b], projv)
  pltpu.sync_copy(Et_hbm.at[b], idxv)
  pltpu.sync_copy(csc_hbm.at[b], outv)

  def s_body(s, _):
    for cc in range(CP // 16):
      sl = pl.ds(cc * 16, 16)
      idx = idxv[pl.ds(s * CP + cc * 16, 16)]
      outv[sl] = outv[sl] + plsc.load_gather(projv, [idx])
    return 0

  lax.fori_loop(0, S, s_body, 0)
  pltpu.sync_copy(outv, out_hbm.at[b])


def kernel(stories, query, E, candidates, A, W, H_w, H_b):
  stories = stories.astype(jnp.int32)
  query = query.astype(jnp.int32)
  E = E.astype(jnp.int32)
  candidates = candidates.astype(jnp.int32)

  segsA = jnp.concatenate(
      [stories.reshape(B * M, S), query.reshape(B, S)], axis=0)
  idxA = jnp.pad(segsA.reshape(NW, SEG_A - 1, S),
                 ((0, 0), (0, 1), (0, 0))).reshape(NW, SEG_A * S)
  idxW = jnp.pad(candidates,
                 ((0, NW * SEG_W - C), (0, 0))).reshape(NW, SEG_W * S)

  outA, outW = _sc_gather(idxA, idxW, A, W)
  flatA = outA[:, :SEG_A - 1, :].reshape(B * M + B, D)
  es = flatA[:B * M].reshape(B, M, D)
  u0 = flatA[B * M:]
  Wc = outW.reshape(NW * SEG_W, D)

  u, csc = _tc_hops(u0, es, H_w, H_b.reshape(1, D), Wc)
  proj = _tc_proj(u, jnp.pad(W, ((0, VP - V), (0, 0))))

  Et = jnp.pad(E.transpose(0, 2, 1),
               ((0, 0), (0, 0), (0, CP - C))).reshape(B, S * CP)
  out = _sc_score(proj, Et, csc)
  return out[:, :C]


# speedup vs baseline: 20.7972x; 1.0921x over previous
import functools

import jax
import jax.numpy as jnp
from jax import lax
from jax.experimental import pallas as pl
from jax.experimental.pallas import tpu as pltpu
from jax.experimental.pallas import tpu_sc as plsc

B, M, S, C, V, D = 32, 50, 20, 1000, 100000, 64
HOPS = 3
NC, NS = 2, 16
NW = NC * NS
SEG_C = 32
CP = 1024
VGRID = 4
VB = V // VGRID
BPG = B // VGRID

_mesh = plsc.VectorSubcoreMesh(
    core_axis_name="c", subcore_axis_name="s", num_cores=NC, num_subcores=NS)
_sc_params = pltpu.CompilerParams(
    use_tc_tiling_on_sc=False, needs_layout_passes=False)


def _wid():
  return lax.axis_index("s") * NC + lax.axis_index("c")


def _seg_reduce(rows_v, o_v, nseg, seg0=0):
  def seg_body(i, _):
    base = i * S
    for d in range(D // 16):
      sl = pl.ds(d * 16, 16)
      acc = rows_v[base, sl]
      for r in range(1, S):
        acc = acc + rows_v[base + r, sl]
      o_v[seg0 + i, sl] = acc
    return 0

  lax.fori_loop(0, nseg, seg_body, 0)


@functools.partial(
    pl.kernel,
    out_type=(
        jax.ShapeDtypeStruct((B, M, D), jnp.float32),
        jax.ShapeDtypeStruct((B, D), jnp.float32),
        jax.ShapeDtypeStruct((C, D), jnp.float32),
    ),
    mesh=_mesh,
    compiler_params=_sc_params,
    scratch_types=[
        pltpu.VMEM((M * S,), jnp.int32),
        pltpu.VMEM((S,), jnp.int32),
        pltpu.VMEM((SEG_C * S,), jnp.int32),
        pltpu.VMEM((M * S + S, D), jnp.float32),
        pltpu.VMEM((M + 1, D), jnp.float32),
        pltpu.VMEM((SEG_C, D), jnp.float32),
        pltpu.SemaphoreType.DMA,
    ],
)
def _sc_gather(stories_hbm, query_hbm, cand_hbm, A_hbm, W_hbm,
               es_hbm, u0_hbm, wc_hbm, sv, qv, cv, rows_v, oa_v, oc_v, sem):
  w = _wid()
  last = NW - 1
  nseg_c = C - last * SEG_C

  pltpu.sync_copy(stories_hbm.at[w], sv)
  pltpu.sync_copy(query_hbm.at[w], qv)

  @pl.when(w == last)
  def _():
    zeros = jnp.zeros((16,), jnp.int32)
    for z in range(nseg_c * S // 16, SEG_C * S // 16):
      cv[pl.ds(z * 16, 16)] = zeros

  @pl.when(w < last)
  def _():
    pltpu.sync_copy(cand_hbm.at[pl.ds(w * SEG_C * S, SEG_C * S)], cv)

  @pl.when(w == last)
  def _():
    pltpu.sync_copy(cand_hbm.at[pl.ds(last * SEG_C * S, nseg_c * S)],
                    cv.at[pl.ds(0, nseg_c * S)])

  a_chunks = [(c * 4, 4) for c in range(M // 4)] + [(M - 2, 2)]
  cps = [
      pltpu.async_copy(A_hbm.at[sv.at[pl.ds(s0 * S, n * S)]],
                       rows_v.at[pl.ds(s0 * S, n * S)], sem)
      for s0, n in a_chunks
  ]
  qcp = pltpu.async_copy(A_hbm.at[qv], rows_v.at[pl.ds(M * S, S)], sem)
  for cp in cps:
    cp.wait()
  qcp.wait()
  _seg_reduce(rows_v, oa_v, M)

  for d in range(D // 16):
    sl = pl.ds(d * 16, 16)
    acc = rows_v[M * S, sl]
    for r in range(1, S):
      acc = acc + rows_v[M * S + r, sl]
    oa_v[M, sl] = acc
  pltpu.sync_copy(oa_v.at[pl.ds(0, M)], es_hbm.at[w])
  pltpu.sync_copy(oa_v.at[M], u0_hbm.at[w])

  wcps = [
      pltpu.async_copy(W_hbm.at[cv.at[pl.ds(c * 4 * S, 4 * S)]],
                       rows_v.at[pl.ds(c * 4 * S, 4 * S)], sem)
      for c in range(SEG_C // 4)
  ]
  for cp in wcps:
    cp.wait()
  nseg = lax.select(w == last, nseg_c, SEG_C)
  _seg_reduce(rows_v, oc_v, nseg)

  @pl.when(w < last)
  def _():
    pltpu.sync_copy(oc_v, wc_hbm.at[pl.ds(w * SEG_C, SEG_C)])

  @pl.when(w == last)
  def _():
    pltpu.sync_copy(oc_v.at[pl.ds(0, nseg_c)],
                    wc_hbm.at[pl.ds(last * SEG_C, nseg_c)])


def _tc_body_full(u0_ref, es_ref, Hw_ref, Hb_ref, Wc_ref, w_ref, e_ref,
                  csc_ref, proj_ref, et_ref, u_sc):
  i = pl.program_id(0)

  @pl.when(i == 0)
  def _():
    u = u0_ref[...]
    es = es_ref[...]
    Hw = Hw_ref[...]
    Hb = Hb_ref[...]
    for _ in range(HOPS):
      sc = jnp.sum(es * u[:, None, :], axis=2)
      sc = sc - jnp.max(sc, axis=1, keepdims=True)
      e = jnp.exp(sc)
      att = e / jnp.sum(e, axis=1, keepdims=True)
      attn = jnp.sum(att[:, :, None] * es, axis=1)
      u = lax.dot_general(u, Hw, (((1,), (1,)), ((), ()))) + Hb + attn
    u_sc[...] = u
    csc_ref[...] = lax.dot_general(u, Wc_ref[...], (((1,), (1,)), ((), ())))

  proj_ref[0] = lax.dot_general(u_sc[...], w_ref[...],
                                (((1,), (1,)), ((), ())))
  et = jnp.transpose(e_ref[...], (0, 2, 1))
  et_ref[...] = jnp.concatenate(
      [et, jnp.zeros((BPG, S, CP - C), jnp.int32)], axis=2)


_tc_stage = pl.pallas_call(
    _tc_body_full,
    grid=(VGRID,),
    in_specs=[
        pl.BlockSpec((B, D), lambda i: (0, 0)),
        pl.BlockSpec((B, M, D), lambda i: (0, 0, 0)),
        pl.BlockSpec((D, D), lambda i: (0, 0)),
        pl.BlockSpec((1, D), lambda i: (0, 0)),
        pl.BlockSpec((C, D), lambda i: (0, 0)),
        pl.BlockSpec((VB, D), lambda i: (i, 0)),
        pl.BlockSpec((BPG, C, S), lambda i: (i, 0, 0)),
    ],
    out_specs=[
        pl.BlockSpec((B, C), lambda i: (0, 0)),
        pl.BlockSpec((1, B, VB), lambda i: (i, 0, 0)),
        pl.BlockSpec((BPG, S, CP), lambda i: (i, 0, 0)),
    ],
    out_shape=[
        jax.ShapeDtypeStruct((B, C), jnp.float32),
        jax.ShapeDtypeStruct((VGRID, B, VB), jnp.float32),
        jax.ShapeDtypeStruct((B, S, CP), jnp.int32),
    ],
    scratch_shapes=[pltpu.VMEM((B, D), jnp.float32)],
)


@functools.partial(
    pl.kernel,
    out_type=jax.ShapeDtypeStruct((B, C), jnp.float32),
    mesh=_mesh,
    compiler_params=_sc_params,
    scratch_types=[
        pltpu.VMEM((V,), jnp.float32),
        pltpu.VMEM((S * CP,), jnp.int32),
        pltpu.VMEM((CP,), jnp.float32),
        pltpu.SemaphoreType.DMA,
    ],
)
def _sc_score(proj_hbm, Et_hbm, csc_hbm, out_hbm, projv, idxv, outv, sem):
  b = _wid()
  for k in range(VGRID):
    pltpu.sync_copy(proj_hbm.at[k, b], projv.at[pl.ds(k * VB, VB)])
  pltpu.sync_copy(Et_hbm.at[b], idxv)
  pltpu.sync_copy(csc_hbm.at[b], outv.at[pl.ds(0, C)])

  def s_body(s, _):
    for cc in range(CP // 16):
      sl = pl.ds(cc * 16, 16)
      idx = idxv[pl.ds(s * CP + cc * 16, 16)]
      outv[sl] = outv[sl] + plsc.load_gather(projv, [idx])
    return 0

  lax.fori_loop(0, S, s_body, 0)
  pltpu.sync_copy(outv.at[pl.ds(0, C)], out_hbm.at[b])


def kernel(stories, query, E, candidates, A, W, H_w, H_b):
  stories = stories.astype(jnp.int32)
  query = query.astype(jnp.int32)
  E = E.astype(jnp.int32)
  candidates = candidates.astype(jnp.int32)

  es, u0, Wc = _sc_gather(
      stories.reshape(B, M * S), query.reshape(B, S),
      candidates.reshape(C * S), A, W)
  csc, proj, Et = _tc_stage(u0, es, H_w, H_b.reshape(1, D), Wc, W, E)
  return _sc_score(proj, Et.reshape(B, S * CP), csc)


# speedup vs baseline: 24.0773x; 1.1577x over previous
import functools

import jax
import jax.numpy as jnp
from jax import lax
from jax.experimental import pallas as pl
from jax.experimental.pallas import tpu as pltpu
from jax.experimental.pallas import tpu_sc as plsc

B, M, S, C, V, D = 32, 50, 20, 1000, 100000, 64
HOPS = 3
NC, NS = 2, 16
NW = NC * NS
CP = 1024
VGRID = 4
VB = V // VGRID
BPG = B // VGRID
CCH = 5

_mesh = plsc.VectorSubcoreMesh(
    core_axis_name="c", subcore_axis_name="s", num_cores=NC, num_subcores=NS)
_sc_params = pltpu.CompilerParams(
    use_tc_tiling_on_sc=False, needs_layout_passes=False)


def _wid():
  return lax.axis_index("s") * NC + lax.axis_index("c")


@functools.partial(
    pl.kernel,
    out_type=(
        jax.ShapeDtypeStruct((B, M, D), jnp.float32),
        jax.ShapeDtypeStruct((B, D), jnp.float32),
    ),
    mesh=_mesh,
    compiler_params=_sc_params,
    scratch_types=[
        pltpu.VMEM((M * S,), jnp.int32),
        pltpu.VMEM((S,), jnp.int32),
        pltpu.VMEM((M * S + S, D), jnp.float32),
        pltpu.VMEM((M + 1, D), jnp.float32),
        pltpu.SemaphoreType.DMA,
    ],
)
def _sc_gather(stories_hbm, query_hbm, A_hbm,
               es_hbm, u0_hbm, sv, qv, rows_v, oa_v, sem):
  w = _wid()
  pltpu.sync_copy(stories_hbm.at[w], sv)
  pltpu.sync_copy(query_hbm.at[w], qv)

  a_chunks = [(c * 4, 4) for c in range(M // 4)] + [(M - 2, 2)]
  cps = [
      pltpu.async_copy(A_hbm.at[sv.at[pl.ds(s0 * S, n * S)]],
                       rows_v.at[pl.ds(s0 * S, n * S)], sem)
      for s0, n in a_chunks
  ]
  qcp = pltpu.async_copy(A_hbm.at[qv], rows_v.at[pl.ds(M * S, S)], sem)
  for cp in cps:
    cp.wait()
  qcp.wait()

  def seg_body(i, _):
    base = i * S
    for d in range(D // 16):
      sl = pl.ds(d * 16, 16)
      acc = rows_v[base, sl]
      for r in range(1, S):
        acc = acc + rows_v[base + r, sl]
      oa_v[i, sl] = acc
    return 0

  lax.fori_loop(0, M + 1, seg_body, 0)
  pltpu.sync_copy(oa_v.at[pl.ds(0, M)], es_hbm.at[w])
  pltpu.sync_copy(oa_v.at[M], u0_hbm.at[w])


def _tc_body(u0_ref, es_ref, Hw_ref, Hb_ref, cand_ref, w_ref, e_ref,
             proj_ref, et_ref, ct_ref, u_sc):
  i = pl.program_id(0)

  @pl.when(i == 0)
  def _():
    u = u0_ref[...]
    es = es_ref[...]
    Hw = Hw_ref[...]
    Hb = Hb_ref[...]
    for _ in range(HOPS):
      sc = jnp.sum(es * u[:, None, :], axis=2)
      sc = sc - jnp.max(sc, axis=1, keepdims=True)
      e = jnp.exp(sc)
      att = e / jnp.sum(e, axis=1, keepdims=True)
      attn = jnp.sum(att[:, :, None] * es, axis=1)
      u = lax.dot_general(u, Hw, (((1,), (1,)), ((), ()))) + Hb + attn
    u_sc[...] = u
    ct = jnp.transpose(cand_ref[...], (1, 0))
    ct_ref[...] = jnp.concatenate(
        [ct, jnp.zeros((S, CP - C), jnp.int32)], axis=1)

  proj_ref[0] = lax.dot_general(u_sc[...], w_ref[...],
                                (((1,), (1,)), ((), ())))
  et = jnp.transpose(e_ref[...], (0, 2, 1))
  et_ref[...] = jnp.concatenate(
      [et, jnp.zeros((BPG, S, CP - C), jnp.int32)], axis=2)


_tc_stage = pl.pallas_call(
    _tc_body,
    grid=(VGRID,),
    in_specs=[
        pl.BlockSpec((B, D), lambda i: (0, 0)),
        pl.BlockSpec((B, M, D), lambda i: (0, 0, 0)),
        pl.BlockSpec((D, D), lambda i: (0, 0)),
        pl.BlockSpec((1, D), lambda i: (0, 0)),
        pl.BlockSpec((C, S), lambda i: (0, 0)),
        pl.BlockSpec((VB, D), lambda i: (i, 0)),
        pl.BlockSpec((BPG, C, S), lambda i: (i, 0, 0)),
    ],
    out_specs=[
        pl.BlockSpec((1, B, VB), lambda i: (i, 0, 0)),
        pl.BlockSpec((BPG, S, CP), lambda i: (i, 0, 0)),
        pl.BlockSpec((S, CP), lambda i: (0, 0)),
    ],
    out_shape=[
        jax.ShapeDtypeStruct((VGRID, B, VB), jnp.float32),
        jax.ShapeDtypeStruct((B, S, CP), jnp.int32),
        jax.ShapeDtypeStruct((S, CP), jnp.int32),
    ],
    scratch_shapes=[pltpu.VMEM((B, D), jnp.float32)],
)


@functools.partial(
    pl.kernel,
    out_type=jax.ShapeDtypeStruct((B, C), jnp.float32),
    mesh=_mesh,
    compiler_params=_sc_params,
    scratch_types=[
        pltpu.VMEM((V,), jnp.float32),
        pltpu.VMEM((S, CP), jnp.int32),
        pltpu.VMEM((CCH, CP), jnp.int32),
        pltpu.VMEM((CP,), jnp.float32),
        pltpu.SemaphoreType.DMA,
    ],
)
def _sc_score(proj_hbm, Et_hbm, Ct_hbm, out_hbm, projv, idxv, ctv, outv, sem):
  b = _wid()
  pcps = [
      pltpu.async_copy(proj_hbm.at[k, b], projv.at[pl.ds(k * VB, VB)], sem)
      for k in range(VGRID)
  ]
  ecp = pltpu.async_copy(Et_hbm.at[b], idxv, sem)
  for cp in pcps:
    cp.wait()
  ecp.wait()

  zeros = jnp.zeros((16,), jnp.float32)
  for cc in range(CP // 16):
    outv[pl.ds(cc * 16, 16)] = zeros

  def gather_rows(rows_ref, s_count):
    def s_body(s, _):
      for cc in range(CP // 16):
        sl = pl.ds(cc * 16, 16)
        idx = rows_ref[s, sl]
        outv[sl] = outv[sl] + plsc.load_gather(projv, [idx])
      return 0
    lax.fori_loop(0, s_count, s_body, 0)

  for q in range(S // CCH):
    pltpu.sync_copy(Ct_hbm.at[pl.ds(q * CCH, CCH)], ctv)
    gather_rows(ctv, CCH)
  gather_rows(idxv, S)

  pltpu.sync_copy(outv.at[pl.ds(0, C)], out_hbm.at[b])


def kernel(stories, query, E, candidates, A, W, H_w, H_b):
  stories = stories.astype(jnp.int32)
  query = query.astype(jnp.int32)
  E = E.astype(jnp.int32)
  candidates = candidates.astype(jnp.int32)

  es, u0 = _sc_gather(stories.reshape(B, M * S), query, A)
  proj, Et, Ct = _tc_stage(u0, es, H_w, H_b.reshape(1, D), candidates, W, E)
  return _sc_score(proj, Et, Ct)
